# Initial kernel scaffold; baseline (speedup 1.0000x reference)
import jax
import jax.numpy as jnp
from jax.experimental import pallas as pl


def kernel(x, token_emb, pos_emb):
    raise NotImplementedError("write your pallas kernel here")


# speedup vs baseline: 7.4697x; 7.4697x over previous
import functools

import jax
import jax.numpy as jnp
from jax import lax
from jax.experimental import pallas as pl
from jax.experimental.pallas import tpu as pltpu
from jax.experimental.pallas import tpu_sc as plsc

_VOCAB = 100000
_MAXLEN = 10000
_D = 64
_B = 4096
_S = 200

_R = _B * _S
_NW = 32
_ROWS_PER_W = _R // _NW
_CHUNK = 128
_NCH = _ROWS_PER_W // _CHUNK
_LANES = 16


def _sc_embed_sum(notes, times, token_emb, pos_emb):
    mesh = plsc.VectorSubcoreMesh(core_axis_name="c", subcore_axis_name="s")

    @functools.partial(
        pl.kernel,
        mesh=mesh,
        out_type=jax.ShapeDtypeStruct((_R, _D), jnp.float32),
        compiler_params=pltpu.CompilerParams(use_tc_tiling_on_sc=False),
        scratch_types=[
            pltpu.VMEM((_NCH, _CHUNK), jnp.int32),
            pltpu.VMEM((_NCH, _CHUNK), jnp.int32),
            pltpu.VMEM((2, _CHUNK, _D), jnp.float32),
            pltpu.VMEM((2, _CHUNK, _D), jnp.float32),
            pltpu.VMEM((2, _CHUNK, _D), jnp.float32),
            pltpu.SemaphoreType.DMA,
            pltpu.SemaphoreType.DMA,
            pltpu.SemaphoreType.DMA,
            pltpu.SemaphoreType.DMA,
            pltpu.SemaphoreType.DMA,
            pltpu.SemaphoreType.DMA,
        ],
    )
    def body(notes_hbm, times_hbm, token_hbm, pos_hbm, out_hbm,
             nidx_v, tidx_v, abuf, bbuf, obuf,
             ga0, ga1, gb0, gb1, so0, so1):
        wid = lax.axis_index("s") * 2 + lax.axis_index("c")
        blk0 = wid * _NCH
        row0 = wid * _ROWS_PER_W
        pltpu.sync_copy(notes_hbm.at[pl.ds(blk0, _NCH)], nidx_v)
        pltpu.sync_copy(times_hbm.at[pl.ds(blk0, _NCH)], tidx_v)

        gas = (ga0, ga1)
        gbs = (gb0, gb1)
        sos = (so0, so1)

        def issue_gathers(j, p):
            pltpu.async_copy(token_hbm.at[nidx_v.at[j]], abuf.at[p], gas[p])
            pltpu.async_copy(pos_hbm.at[tidx_v.at[j]], bbuf.at[p], gbs[p])

        def wait_gathers(j, p):
            pltpu.make_async_copy(
                token_hbm.at[nidx_v.at[j]], abuf.at[p], gas[p]).wait()
            pltpu.make_async_copy(
                pos_hbm.at[tidx_v.at[j]], bbuf.at[p], gbs[p]).wait()

        def add(p):
            a, b, o = abuf.at[p], bbuf.at[p], obuf.at[p]

            @plsc.parallel_loop(0, _CHUNK, step=1, unroll=4)
            def _(r):
                for k in range(0, _D, _LANES):
                    sl = pl.ds(k, _LANES)
                    o[r, sl] = a[r, sl] + b[r, sl]

        def issue_scatter(j, p):
            pltpu.async_copy(
                obuf.at[p], out_hbm.at[pl.ds(row0 + j * _CHUNK, _CHUNK)],
                sos[p])

        def wait_scatter(p):
            pltpu.make_async_copy(
                obuf.at[p], out_hbm.at[pl.ds(row0, _CHUNK)], sos[p]).wait()

        issue_gathers(0, 0)
        issue_gathers(1, 1)
        wait_gathers(0, 0)
        add(0)
        issue_gathers(2, 0)
        issue_scatter(0, 0)
        wait_gathers(1, 1)
        add(1)
        issue_gathers(3, 1)
        issue_scatter(1, 1)

        def loop_body(g, carry):
            j0 = 2 * g
            wait_gathers(j0, 0)
            wait_scatter(0)
            add(0)
            issue_gathers(j0 + 2, 0)
            issue_scatter(j0, 0)
            j1 = j0 + 1
            wait_gathers(j1, 1)
            wait_scatter(1)
            add(1)
            issue_gathers(j1 + 2, 1)
            issue_scatter(j1, 1)
            return carry

        lax.fori_loop(1, _NCH // 2 - 1, loop_body, 0)

        wait_gathers(_NCH - 2, 0)
        wait_scatter(0)
        add(0)
        issue_scatter(_NCH - 2, 0)
        wait_gathers(_NCH - 1, 1)
        wait_scatter(1)
        add(1)
        issue_scatter(_NCH - 1, 1)
        wait_scatter(0)
        wait_scatter(1)

    return body(notes, times, token_emb, pos_emb)


@jax.jit
def kernel(x, token_emb, pos_emb):
    notes = x[:, 0, :].astype(jnp.int32).reshape(_R // _CHUNK, _CHUNK)
    times = x[:, 1, :].astype(jnp.int32).reshape(_R // _CHUNK, _CHUNK)
    out = _sc_embed_sum(notes, times, token_emb, pos_emb)
    return out.reshape(_B, _S, _D)


# speedup vs baseline: 7.8152x; 1.0462x over previous
import functools

import jax
import jax.numpy as jnp
from jax import lax
from jax.experimental import pallas as pl
from jax.experimental.pallas import tpu as pltpu
from jax.experimental.pallas import tpu_sc as plsc

_VOCAB = 100000
_MAXLEN = 10000
_D = 64
_B = 4096
_S = 200

_R = _B * _S
_NW = 32
_ROWS_PER_W = _R // _NW
_CHUNK = 128
_NCH = _ROWS_PER_W // _CHUNK
_NBLK = _R // _CHUNK
_LANES = 16


def _sc_embed_sum(notes, times, token_emb, pos_emb):
    mesh = plsc.VectorSubcoreMesh(core_axis_name="c", subcore_axis_name="s")

    @functools.partial(
        pl.kernel,
        mesh=mesh,
        out_type=jax.ShapeDtypeStruct((_R, _D), jnp.float32),
        compiler_params=pltpu.CompilerParams(use_tc_tiling_on_sc=False),
        scratch_types=[
            pltpu.VMEM((4, _CHUNK), jnp.int32),
            pltpu.VMEM((4, _CHUNK), jnp.int32),
            pltpu.VMEM((2, _CHUNK, _D), jnp.float32),
            pltpu.VMEM((2, _CHUNK, _D), jnp.float32),
            pltpu.VMEM((2, _CHUNK, _D), jnp.float32),
            pltpu.VMEM_SHARED((_MAXLEN, _D), jnp.float32),
            pltpu.SemaphoreType.DMA,
            pltpu.SemaphoreType.DMA,
            pltpu.SemaphoreType.DMA,
            pltpu.SemaphoreType.DMA,
            pltpu.SemaphoreType.DMA,
            pltpu.SemaphoreType.DMA,
            pltpu.SemaphoreType.DMA,
            pltpu.SemaphoreType.DMA,
            pltpu.SemaphoreType.DMA,
            pltpu.SemaphoreType.DMA,
        ],
    )
    def body(notes_hbm, times_hbm, token_hbm, pos_hbm, out_hbm,
             nidx_v, tidx_v, abuf, bbuf, obuf, pos_sh,
             gi0, gi1, gi2, gi3, ga0, ga1, gb0, gb1, so0, so1):
        sid = lax.axis_index("s")
        wid = sid * 2 + lax.axis_index("c")
        blk0 = wid * _NCH
        row0 = wid * _ROWS_PER_W

        _TROWS = _MAXLEN // 16
        tslice = pl.ds(sid * _TROWS, _TROWS)
        pltpu.sync_copy(pos_hbm.at[tslice], pos_sh.at[tslice])
        plsc.subcore_barrier()

        gis = (gi0, gi1, gi2, gi3)
        gas = (ga0, ga1)
        gbs = (gb0, gb1)
        sos = (so0, so1)

        def idx_row(j):
            return jnp.minimum(blk0 + j, _NBLK - 1)

        def issue_idx(j, s):
            pltpu.async_copy(notes_hbm.at[idx_row(j)], nidx_v.at[s], gis[s])
            pltpu.async_copy(times_hbm.at[idx_row(j)], tidx_v.at[s], gis[s])

        def wait_idx(j, s):
            pltpu.make_async_copy(
                notes_hbm.at[idx_row(j)], nidx_v.at[s], gis[s]).wait()
            pltpu.make_async_copy(
                times_hbm.at[idx_row(j)], tidx_v.at[s], gis[s]).wait()

        def issue_gathers(s, p):
            pltpu.async_copy(token_hbm.at[nidx_v.at[s]], abuf.at[p], gas[p])
            pltpu.async_copy(pos_sh.at[tidx_v.at[s]], bbuf.at[p], gbs[p])

        def wait_gathers(s, p):
            pltpu.make_async_copy(
                token_hbm.at[nidx_v.at[s]], abuf.at[p], gas[p]).wait()
            pltpu.make_async_copy(
                pos_sh.at[tidx_v.at[s]], bbuf.at[p], gbs[p]).wait()

        def add(p):
            a, b, o = abuf.at[p], bbuf.at[p], obuf.at[p]

            @plsc.parallel_loop(0, _CHUNK, step=1, unroll=4)
            def _(r):
                for k in range(0, _D, _LANES):
                    sl = pl.ds(k, _LANES)
                    o[r, sl] = a[r, sl] + b[r, sl]

        def issue_scatter(j, p):
            pltpu.async_copy(
                obuf.at[p], out_hbm.at[pl.ds(row0 + j * _CHUNK, _CHUNK)],
                sos[p])

        def wait_scatter(p):
            pltpu.make_async_copy(
                obuf.at[p], out_hbm.at[pl.ds(row0, _CHUNK)], sos[p]).wait()

        def step(j, s, p, first):
            wait_gathers(s, p)
            issue_idx(j + 4, s)
            if not first:
                wait_scatter(p)
            add(p)
            wait_idx(j + 2, (s + 2) % 4)
            issue_gathers((s + 2) % 4, p)
            issue_scatter(j, p)

        for k in range(4):
            issue_idx(k, k)
        for k in range(2):
            wait_idx(k, k)
            issue_gathers(k, k)
        step(0, 0, 0, True)
        step(1, 1, 1, True)

        def loop_body(q, carry):
            j = 4 * q + 2
            step(j, 2, 0, False)
            step(j + 1, 3, 1, False)
            step(j + 2, 0, 0, False)
            step(j + 3, 1, 1, False)
            return carry

        lax.fori_loop(0, (_NCH - 4) // 4, loop_body, 0)

        for j, s, p in ((_NCH - 2, 2, 0), (_NCH - 1, 3, 1)):
            wait_gathers(s, p)
            wait_scatter(p)
            add(p)
            issue_scatter(j, p)
        wait_scatter(0)
        wait_scatter(1)
        wait_idx(_NCH + 0, 0)
        wait_idx(_NCH + 1, 1)

    return body(notes, times, token_emb, pos_emb)


@jax.jit
def kernel(x, token_emb, pos_emb):
    notes = x[:, 0, :].astype(jnp.int32).reshape(_NBLK, _CHUNK)
    times = x[:, 1, :].astype(jnp.int32).reshape(_NBLK, _CHUNK)
    out = _sc_embed_sum(notes, times, token_emb, pos_emb)
    return out.reshape(_B, _S, _D)
